# baseline (speedup 1.0000x reference)
import functools

import jax
import jax.numpy as jnp
from jax import lax
from jax.experimental import pallas as pl
from jax.experimental.pallas import tpu as pltpu
from jax.experimental.pallas import tpu_sc as plsc

N_NODES = 10000
N_EDGES = 320000
NODE_NF = 11
EDGE_NF = 4
H_NF = 128
EMB_NF = 4

NC = 2
NS = 16
NW = NC * NS

CH = 128
EPW = 10240
NCH = EPW // CH
E_PAD = EPW * NW
N_PAD = 10112
RPT = N_PAD // NS

_F32 = jnp.float32


def _sc_mesh():
    return plsc.VectorSubcoreMesh(
        core_axis_name="c", subcore_axis_name="s", num_cores=NC, num_subcores=NS
    )


RG = 6
GLEAD = 4


def _gather_call(nf16, row3, col3, ea16):
    @functools.partial(
        pl.kernel,
        out_type=jax.ShapeDtypeStruct((E_PAD, 48), _F32),
        mesh=_sc_mesh(),
        scratch_types=[
            pltpu.VMEM((NCH, CH), jnp.int32),
            pltpu.VMEM((NCH, CH), jnp.int32),
            pltpu.VMEM((RG, CH, 16), _F32),
            pltpu.VMEM((RG, CH, 16), _F32),
            pltpu.VMEM((RG, CH, 16), _F32),
            pltpu.SemaphoreType.DMA,
            pltpu.SemaphoreType.DMA,
            pltpu.SemaphoreType.DMA,
            pltpu.SemaphoreType.DMA,
        ],
        compiler_params=pltpu.CompilerParams(use_tc_tiling_on_sc=False),
    )
    def k(nf_hbm, row_hbm, col_hbm, ea_hbm, g_hbm,
          ridx2, cidx2, sbuf, dbuf, abuf, gsem_r, gsem_c, gsem_a, wsem):
        wid = lax.axis_index("c") * NS + lax.axis_index("s")

        pltpu.sync_copy(row_hbm.at[wid], ridx2)
        pltpu.sync_copy(col_hbm.at[wid], cidx2)

        def ebase(t):
            return pl.multiple_of(wid * EPW + t * CH, CH)

        def start_gather(t, b):
            pltpu.async_copy(nf_hbm.at[ridx2.at[t]], sbuf.at[b], gsem_r)
            pltpu.async_copy(nf_hbm.at[cidx2.at[t]], dbuf.at[b], gsem_c)
            pltpu.async_copy(ea_hbm.at[pl.ds(ebase(t), CH)], abuf.at[b], gsem_a)

        def wait_gather(t, b):
            pltpu.make_async_copy(nf_hbm.at[ridx2.at[t]], sbuf.at[b], gsem_r).wait()
            pltpu.make_async_copy(nf_hbm.at[cidx2.at[t]], dbuf.at[b], gsem_c).wait()
            pltpu.make_async_copy(ea_hbm.at[pl.ds(ebase(t), CH)], abuf.at[b],
                                  gsem_a).wait()

        def out_slices(t):
            base = ebase(t)
            return (g_hbm.at[pl.ds(base, CH), pl.ds(0, 16)],
                    g_hbm.at[pl.ds(base, CH), pl.ds(16, 16)],
                    g_hbm.at[pl.ds(base, CH), pl.ds(32, 16)])

        def start_write(t, b):
            o_s, o_d, o_a = out_slices(t)
            pltpu.async_copy(sbuf.at[b], o_s, wsem)
            pltpu.async_copy(dbuf.at[b], o_d, wsem)
            pltpu.async_copy(abuf.at[b], o_a, wsem)

        def wait_write(t, b):
            o_s, o_d, o_a = out_slices(t)
            pltpu.make_async_copy(sbuf.at[b], o_s, wsem).wait()
            pltpu.make_async_copy(dbuf.at[b], o_d, wsem).wait()
            pltpu.make_async_copy(abuf.at[b], o_a, wsem).wait()

        for t in range(GLEAD):
            start_gather(t, t % RG)

        def body(g, carry):
            for b_off in range(RG):
                t = g * RG + b_off
                b = b_off
                bw = (b_off - 2) % RG

                @pl.when(t >= 2)
                def _():
                    wait_write(t - 2, bw)

                @pl.when(t + GLEAD < NCH)
                def _():
                    start_gather(t + GLEAD, bw)

                wait_gather(t, b)
                start_write(t, b)
            return carry

        lax.fori_loop(0, NCH // RG, body, 0, unroll=False)
        for t in range(NCH - NCH % RG, NCH):
            b = t % RG
            bw = (b - 2) % RG
            wait_write(t - 2, bw)

            @pl.when(t + GLEAD < NCH)
            def _():
                start_gather(t + GLEAD, bw)

            wait_gather(t, b)
            start_write(t, b)
        wait_write(NCH - 2, (NCH - 2) % RG)
        wait_write(NCH - 1, (NCH - 1) % RG)

    return k(nf16, row3, col3, ea16)


RS = 2
SLEAD = 1


def _scatter_call(ef, row3, zeros_big):
    @functools.partial(
        pl.kernel,
        out_type=(
            jax.ShapeDtypeStruct((N_PAD, H_NF), _F32),
            jax.ShapeDtypeStruct((N_PAD, H_NF), _F32),
        ),
        mesh=_sc_mesh(),
        scratch_types=[
            pltpu.VMEM((NCH, CH), jnp.int32),
            pltpu.VMEM((RS, CH, H_NF), _F32),
            pltpu.VMEM_SHARED((N_PAD, H_NF), _F32),
            pltpu.SemaphoreType.DMA,
            pltpu.SemaphoreType.DMA,
        ],
        compiler_params=pltpu.CompilerParams(use_tc_tiling_on_sc=False),
    )
    def k(ef_hbm, row_hbm, z_hbm, p0_hbm, p1_hbm, idx2, ebuf, agg_sh,
          lsem, asem):
        c = lax.axis_index("c")
        s = lax.axis_index("s")
        wid = c * NS + s
        rslice = pl.ds(s * RPT, RPT)
        pltpu.sync_copy(z_hbm.at[rslice], agg_sh.at[rslice])
        pltpu.sync_copy(row_hbm.at[wid], idx2)
        plsc.subcore_barrier()

        def ef_slice(t):
            base = pl.multiple_of(wid * EPW + t * CH, CH)
            return ef_hbm.at[pl.ds(base, CH)]

        def start_load(t, b):
            pltpu.async_copy(ef_slice(t), ebuf.at[b], lsem)

        def wait_load(t, b):
            pltpu.make_async_copy(ef_slice(t), ebuf.at[b], lsem).wait()

        def start_add(t, b):
            pltpu.async_copy(ebuf.at[b], agg_sh.at[idx2.at[t]], asem, add=True)

        def wait_add(t, b):
            pltpu.make_async_copy(ebuf.at[b], agg_sh.at[idx2.at[t]], asem).wait()

        start_load(0, 0)

        def body(g, carry):
            for b in range(RS):
                t = g * RS + b
                bo = 1 - b

                @pl.when(t >= 1)
                def _():
                    wait_add(t - 1, bo)

                @pl.when(t + 1 < NCH)
                def _():
                    start_load(t + 1, bo)

                wait_load(t, b)
                start_add(t, b)
            return carry

        lax.fori_loop(0, NCH // RS, body, 0, unroll=False)
        wait_add(NCH - 1, (NCH - 1) % RS)
        plsc.subcore_barrier()

        @pl.when(c == 0)
        def _():
            pltpu.sync_copy(agg_sh.at[rslice], p0_hbm.at[rslice])

        @pl.when(c == 1)
        def _():
            pltpu.sync_copy(agg_sh.at[rslice], p1_hbm.at[rslice])

    return k(ef, row3, zeros_big)


BE = 2048


_BF16 = jnp.bfloat16


def _edge_mlp_kernel(x, w1all, b1, w2, b2, out):
    h = jnp.dot(x[...].astype(_BF16), w1all[...], preferred_element_type=_F32)
    h = jnp.maximum(h + b1[...], 0.0)
    h = jnp.dot(h.astype(_BF16), w2[...], preferred_element_type=_F32) + b2[...]
    out[...] = jnp.maximum(h, 0.0)


def _edge_mlp_call(g48, w1all, b1, w2, b2):
    grid = (E_PAD // BE,)
    bcast = lambda shape: pl.BlockSpec(shape, lambda i: (0, 0))
    return pl.pallas_call(
        _edge_mlp_kernel,
        grid=grid,
        in_specs=[
            pl.BlockSpec((BE, 48), lambda i: (i, 0)),
            bcast((48, H_NF)),
            bcast((1, H_NF)),
            bcast((H_NF, H_NF)),
            bcast((1, H_NF)),
        ],
        out_specs=pl.BlockSpec((BE, H_NF), lambda i: (i, 0)),
        out_shape=jax.ShapeDtypeStruct((E_PAD, H_NF), _F32),
    )(g48, w1all, b1, w2, b2)


BN = 1024


def _node_mlp_kernel(nf, p0, p1, w1n, w1a, b1, w2, b2, fw, fb, out):
    agg = p0[...] + p1[...]
    h = jnp.dot(nf[...], w1n[...], preferred_element_type=_F32)
    h = h + jnp.dot(agg, w1a[...], preferred_element_type=_F32)
    h = jnp.maximum(h + b1[...], 0.0)
    h = jnp.dot(h, w2[...], preferred_element_type=_F32) + b2[...]
    out[...] = jnp.dot(h, fw[...], preferred_element_type=_F32) + fb[...]


def _node_mlp_call(nf16, p0, p1, w1n, w1a, b1, w2, b2, fw8, fb8):
    grid = (pl.cdiv(N_NODES, BN),)
    bcast = lambda shape: pl.BlockSpec(shape, lambda i: (0, 0))
    return pl.pallas_call(
        _node_mlp_kernel,
        grid=grid,
        in_specs=[
            pl.BlockSpec((BN, 16), lambda i: (i, 0)),
            pl.BlockSpec((BN, H_NF), lambda i: (i, 0)),
            pl.BlockSpec((BN, H_NF), lambda i: (i, 0)),
            bcast((16, H_NF)),
            bcast((H_NF, H_NF)),
            bcast((1, H_NF)),
            bcast((H_NF, H_NF)),
            bcast((1, H_NF)),
            bcast((H_NF, 8)),
            bcast((1, 8)),
        ],
        out_specs=pl.BlockSpec((BN, 8), lambda i: (i, 0)),
        out_shape=jax.ShapeDtypeStruct((N_NODES, 8), _F32),
    )(nf16, p0, p1, w1n, w1a, b1, w2, b2, fw8, fb8)


def kernel(node_feats, edge_index, edge_attr,
           eW1, eb1, eW2, eb2, nW1, nb1, nW2, nb2, fW, fb):
    row = edge_index[0]
    col = edge_index[1]
    pad_idx = jnp.full((E_PAD - N_EDGES,), N_NODES, jnp.int32)
    row3 = jnp.concatenate([row, pad_idx]).reshape(NW, NCH, CH)
    col3 = jnp.concatenate([col, pad_idx]).reshape(NW, NCH, CH)

    nf16 = jnp.zeros((N_PAD, 16), _F32).at[:N_NODES, :NODE_NF].set(node_feats)
    ea16 = jnp.zeros((E_PAD, 16), _F32).at[:N_EDGES, :EDGE_NF].set(edge_attr)

    w1all = jnp.zeros((48, H_NF), _F32)
    w1all = w1all.at[:NODE_NF].set(eW1[:NODE_NF])
    w1all = w1all.at[16:16 + NODE_NF].set(eW1[NODE_NF:2 * NODE_NF])
    w1all = w1all.at[32:32 + EDGE_NF].set(eW1[2 * NODE_NF:])
    e_b1 = eb1.reshape(1, H_NF)
    e_b2 = eb2.reshape(1, H_NF)

    w1n = jnp.zeros((16, H_NF), _F32).at[:NODE_NF].set(nW1[:NODE_NF])
    w1a = nW1[NODE_NF:]
    n_b1 = nb1.reshape(1, H_NF)
    n_b2 = nb2.reshape(1, H_NF)
    fw8 = jnp.zeros((H_NF, 8), _F32).at[:, :EMB_NF].set(fW)
    fb8 = jnp.zeros((1, 8), _F32).at[0, :EMB_NF].set(fb)

    g48 = _gather_call(nf16, row3, col3, ea16)
    ef = _edge_mlp_call(g48, w1all.astype(_BF16), e_b1,
                        eW2.astype(_BF16), e_b2)
    zeros_big = jnp.zeros((N_PAD, H_NF), _F32)
    p0, p1 = _scatter_call(ef, row3, zeros_big)
    out8 = _node_mlp_call(nf16, p0, p1, w1n, w1a, n_b1, nW2, n_b2, fw8, fb8)
    return out8[:, :EMB_NF]


# speedup vs baseline: 1.5009x; 1.5009x over previous
import functools

import jax
import jax.numpy as jnp
from jax import lax
from jax.experimental import pallas as pl
from jax.experimental.pallas import tpu as pltpu
from jax.experimental.pallas import tpu_sc as plsc

N_NODES = 10000
N_EDGES = 320000
NODE_NF = 11
EDGE_NF = 4
H_NF = 128
EMB_NF = 4

NC = 2
NS = 16
NW = NC * NS

CH = 128
EPW = 10240
NCH = EPW // CH
E_PAD = EPW * NW
N_PAD = 10112
RPT = N_PAD // NS

_F32 = jnp.float32


def _sc_mesh():
    return plsc.VectorSubcoreMesh(
        core_axis_name="c", subcore_axis_name="s", num_cores=NC, num_subcores=NS
    )


RG = 6
GLEAD = 4


def _gather_call(nf16, row3, col3):
    @functools.partial(
        pl.kernel,
        out_type=jax.ShapeDtypeStruct((E_PAD, 32), _F32),
        mesh=_sc_mesh(),
        scratch_types=[
            pltpu.VMEM((NCH, CH), jnp.int32),
            pltpu.VMEM((NCH, CH), jnp.int32),
            pltpu.VMEM((RG, CH, 16), _F32),
            pltpu.VMEM((RG, CH, 16), _F32),
            pltpu.SemaphoreType.DMA,
            pltpu.SemaphoreType.DMA,
            pltpu.SemaphoreType.DMA,
        ],
        compiler_params=pltpu.CompilerParams(use_tc_tiling_on_sc=False),
    )
    def k(nf_hbm, row_hbm, col_hbm, g_hbm,
          ridx2, cidx2, sbuf, dbuf, gsem_r, gsem_c, wsem):
        wid = lax.axis_index("c") * NS + lax.axis_index("s")

        pltpu.sync_copy(row_hbm.at[wid], ridx2)
        pltpu.sync_copy(col_hbm.at[wid], cidx2)

        def ebase(t):
            return pl.multiple_of(wid * EPW + t * CH, CH)

        def start_gather(t, b):
            pltpu.async_copy(nf_hbm.at[ridx2.at[t]], sbuf.at[b], gsem_r)
            pltpu.async_copy(nf_hbm.at[cidx2.at[t]], dbuf.at[b], gsem_c)

        def wait_gather(t, b):
            pltpu.make_async_copy(nf_hbm.at[ridx2.at[t]], sbuf.at[b], gsem_r).wait()
            pltpu.make_async_copy(nf_hbm.at[cidx2.at[t]], dbuf.at[b], gsem_c).wait()

        def out_slices(t):
            base = ebase(t)
            return (g_hbm.at[pl.ds(base, CH), pl.ds(0, 16)],
                    g_hbm.at[pl.ds(base, CH), pl.ds(16, 16)])

        def start_write(t, b):
            o_s, o_d = out_slices(t)
            pltpu.async_copy(sbuf.at[b], o_s, wsem)
            pltpu.async_copy(dbuf.at[b], o_d, wsem)

        def wait_write(t, b):
            o_s, o_d = out_slices(t)
            pltpu.make_async_copy(sbuf.at[b], o_s, wsem).wait()
            pltpu.make_async_copy(dbuf.at[b], o_d, wsem).wait()

        for t in range(GLEAD):
            start_gather(t, t % RG)

        def body(g, carry):
            for b_off in range(RG):
                t = g * RG + b_off
                b = b_off
                bw = (b_off - 2) % RG

                @pl.when(t >= 2)
                def _():
                    wait_write(t - 2, bw)

                @pl.when(t + GLEAD < NCH)
                def _():
                    start_gather(t + GLEAD, bw)

                wait_gather(t, b)
                start_write(t, b)
            return carry

        lax.fori_loop(0, NCH // RG, body, 0, unroll=False)
        for t in range(NCH - NCH % RG, NCH):
            b = t % RG
            bw = (b - 2) % RG
            wait_write(t - 2, bw)

            @pl.when(t + GLEAD < NCH)
            def _():
                start_gather(t + GLEAD, bw)

            wait_gather(t, b)
            start_write(t, b)
        wait_write(NCH - 2, (NCH - 2) % RG)
        wait_write(NCH - 1, (NCH - 1) % RG)

    return k(nf16, row3, col3)


RS = 2
SLEAD = 1


def _scatter_call(ef, row3, zeros_big):
    @functools.partial(
        pl.kernel,
        out_type=(
            jax.ShapeDtypeStruct((N_PAD, H_NF), _F32),
            jax.ShapeDtypeStruct((N_PAD, H_NF), _F32),
        ),
        mesh=_sc_mesh(),
        scratch_types=[
            pltpu.VMEM((NCH, CH), jnp.int32),
            pltpu.VMEM((RS, CH, H_NF), _F32),
            pltpu.VMEM_SHARED((N_PAD, H_NF), _F32),
            pltpu.SemaphoreType.DMA,
            pltpu.SemaphoreType.DMA,
        ],
        compiler_params=pltpu.CompilerParams(use_tc_tiling_on_sc=False),
    )
    def k(ef_hbm, row_hbm, z_hbm, p0_hbm, p1_hbm, idx2, ebuf, agg_sh,
          lsem, asem):
        c = lax.axis_index("c")
        s = lax.axis_index("s")
        wid = c * NS + s
        rslice = pl.ds(s * RPT, RPT)
        pltpu.sync_copy(z_hbm.at[rslice], agg_sh.at[rslice])
        pltpu.sync_copy(row_hbm.at[wid], idx2)
        plsc.subcore_barrier()

        def ef_slice(t):
            base = pl.multiple_of(wid * EPW + t * CH, CH)
            return ef_hbm.at[pl.ds(base, CH)]

        def start_load(t, b):
            pltpu.async_copy(ef_slice(t), ebuf.at[b], lsem)

        def wait_load(t, b):
            pltpu.make_async_copy(ef_slice(t), ebuf.at[b], lsem).wait()

        def start_add(t, b):
            pltpu.async_copy(ebuf.at[b], agg_sh.at[idx2.at[t]], asem, add=True)

        def wait_add(t, b):
            pltpu.make_async_copy(ebuf.at[b], agg_sh.at[idx2.at[t]], asem).wait()

        start_load(0, 0)

        def body(g, carry):
            for b in range(RS):
                t = g * RS + b
                bo = 1 - b

                @pl.when(t >= 1)
                def _():
                    wait_add(t - 1, bo)

                @pl.when(t + 1 < NCH)
                def _():
                    start_load(t + 1, bo)

                wait_load(t, b)
                start_add(t, b)
            return carry

        lax.fori_loop(0, NCH // RS, body, 0, unroll=False)
        wait_add(NCH - 1, (NCH - 1) % RS)
        plsc.subcore_barrier()

        @pl.when(c == 0)
        def _():
            pltpu.sync_copy(agg_sh.at[rslice], p0_hbm.at[rslice])

        @pl.when(c == 1)
        def _():
            pltpu.sync_copy(agg_sh.at[rslice], p1_hbm.at[rslice])

    return k(ef, row3, zeros_big)


BE = 2048


_BF16 = jnp.bfloat16


def _edge_mlp_kernel(x, eaT, w1sd, w1e, b1, w2, b2, out):
    h = jnp.dot(x[...].astype(_BF16), w1sd[...], preferred_element_type=_F32)
    h = h + lax.dot_general(eaT[...].astype(_BF16), w1e[...],
                            (((0,), (0,)), ((), ())),
                            preferred_element_type=_F32)
    h = jnp.maximum(h + b1[...], 0.0)
    h = jnp.dot(h.astype(_BF16), w2[...], preferred_element_type=_F32) + b2[...]
    out[...] = jnp.maximum(h, 0.0)


def _edge_mlp_call(g32, eaT, w1sd, w1e, b1, w2, b2):
    grid = (E_PAD // BE,)
    bcast = lambda shape: pl.BlockSpec(shape, lambda i: (0, 0))
    return pl.pallas_call(
        _edge_mlp_kernel,
        grid=grid,
        in_specs=[
            pl.BlockSpec((BE, 32), lambda i: (i, 0)),
            pl.BlockSpec((EDGE_NF, BE), lambda i: (0, i)),
            bcast((32, H_NF)),
            bcast((EDGE_NF, H_NF)),
            bcast((1, H_NF)),
            bcast((H_NF, H_NF)),
            bcast((1, H_NF)),
        ],
        out_specs=pl.BlockSpec((BE, H_NF), lambda i: (i, 0)),
        out_shape=jax.ShapeDtypeStruct((E_PAD, H_NF), _F32),
    )(g32, eaT, w1sd, w1e, b1, w2, b2)


BN = 1024


def _node_mlp_kernel(nf, p0, p1, w1n, w1a, b1, w2, b2, fw, fb, out):
    agg = p0[...] + p1[...]
    h = jnp.dot(nf[...], w1n[...], preferred_element_type=_F32)
    h = h + jnp.dot(agg, w1a[...], preferred_element_type=_F32)
    h = jnp.maximum(h + b1[...], 0.0)
    h = jnp.dot(h, w2[...], preferred_element_type=_F32) + b2[...]
    out[...] = jnp.dot(h, fw[...], preferred_element_type=_F32) + fb[...]


def _node_mlp_call(nf16, p0, p1, w1n, w1a, b1, w2, b2, fw8, fb8):
    grid = (pl.cdiv(N_NODES, BN),)
    bcast = lambda shape: pl.BlockSpec(shape, lambda i: (0, 0))
    return pl.pallas_call(
        _node_mlp_kernel,
        grid=grid,
        in_specs=[
            pl.BlockSpec((BN, 16), lambda i: (i, 0)),
            pl.BlockSpec((BN, H_NF), lambda i: (i, 0)),
            pl.BlockSpec((BN, H_NF), lambda i: (i, 0)),
            bcast((16, H_NF)),
            bcast((H_NF, H_NF)),
            bcast((1, H_NF)),
            bcast((H_NF, H_NF)),
            bcast((1, H_NF)),
            bcast((H_NF, 8)),
            bcast((1, 8)),
        ],
        out_specs=pl.BlockSpec((BN, 8), lambda i: (i, 0)),
        out_shape=jax.ShapeDtypeStruct((N_NODES, 8), _F32),
    )(nf16, p0, p1, w1n, w1a, b1, w2, b2, fw8, fb8)


def kernel(node_feats, edge_index, edge_attr,
           eW1, eb1, eW2, eb2, nW1, nb1, nW2, nb2, fW, fb):
    row = edge_index[0]
    col = edge_index[1]
    pad_idx = jnp.full((E_PAD - N_EDGES,), N_NODES, jnp.int32)
    row3 = jnp.concatenate([row, pad_idx]).reshape(NW, NCH, CH)
    col3 = jnp.concatenate([col, pad_idx]).reshape(NW, NCH, CH)

    nf16 = jnp.zeros((N_PAD, 16), _F32).at[:N_NODES, :NODE_NF].set(node_feats)
    eaT = jnp.pad(edge_attr.T, ((0, 0), (0, E_PAD - N_EDGES)))

    w1sd = jnp.zeros((32, H_NF), _F32)
    w1sd = w1sd.at[:NODE_NF].set(eW1[:NODE_NF])
    w1sd = w1sd.at[16:16 + NODE_NF].set(eW1[NODE_NF:2 * NODE_NF])
    w1e = eW1[2 * NODE_NF:]
    e_b1 = eb1.reshape(1, H_NF)
    e_b2 = eb2.reshape(1, H_NF)

    w1n = jnp.zeros((16, H_NF), _F32).at[:NODE_NF].set(nW1[:NODE_NF])
    w1a = nW1[NODE_NF:]
    n_b1 = nb1.reshape(1, H_NF)
    n_b2 = nb2.reshape(1, H_NF)
    fw8 = jnp.zeros((H_NF, 8), _F32).at[:, :EMB_NF].set(fW)
    fb8 = jnp.zeros((1, 8), _F32).at[0, :EMB_NF].set(fb)

    g32 = _gather_call(nf16, row3, col3)
    ef = _edge_mlp_call(g32, eaT, w1sd.astype(_BF16), w1e.astype(_BF16),
                        e_b1, eW2.astype(_BF16), e_b2)
    zeros_big = jnp.zeros((N_PAD, H_NF), _F32)
    p0, p1 = _scatter_call(ef, row3, zeros_big)
    out8 = _node_mlp_call(nf16, p0, p1, w1n, w1a, n_b1, nW2, n_b2, fw8, fb8)
    return out8[:, :EMB_NF]


# speedup vs baseline: 1.9343x; 1.2888x over previous
import functools

import jax
import jax.numpy as jnp
from jax import lax
from jax.experimental import pallas as pl
from jax.experimental.pallas import tpu as pltpu
from jax.experimental.pallas import tpu_sc as plsc

N_NODES = 10000
N_EDGES = 320000
NODE_NF = 11
EDGE_NF = 4
H_NF = 128
EMB_NF = 4

NC = 2
NS = 16
NW = NC * NS

CH = 128
EPW = 10240
NCH = EPW // CH
E_PAD = EPW * NW
N_PAD = 10112
RPT = N_PAD // NS

_F32 = jnp.float32


def _sc_mesh():
    return plsc.VectorSubcoreMesh(
        core_axis_name="c", subcore_axis_name="s", num_cores=NC, num_subcores=NS
    )


RG = 6
GLEAD = 4


def _gather_call(nf16, row3, col3, eaT):
    @functools.partial(
        pl.kernel,
        out_type=jax.ShapeDtypeStruct((E_PAD, 32), _F32),
        mesh=_sc_mesh(),
        scratch_types=[
            pltpu.VMEM((NCH, CH), jnp.int32),
            pltpu.VMEM((NCH, CH), jnp.int32),
            pltpu.VMEM((RG, CH, 16), _F32),
            pltpu.VMEM((RG, CH, 16), _F32),
            pltpu.VMEM((RG, EDGE_NF, CH), _F32),
            pltpu.SemaphoreType.DMA,
            pltpu.SemaphoreType.DMA,
            pltpu.SemaphoreType.DMA,
            pltpu.SemaphoreType.DMA,
        ],
        compiler_params=pltpu.CompilerParams(use_tc_tiling_on_sc=False,
                                             needs_layout_passes=False),
    )
    def k(nf_hbm, row_hbm, col_hbm, ea_hbm, g_hbm,
          ridx2, cidx2, sbuf, dbuf, abuf, gsem_r, gsem_c, gsem_a, wsem):
        wid = lax.axis_index("c") * NS + lax.axis_index("s")

        pltpu.sync_copy(row_hbm.at[wid], ridx2)
        pltpu.sync_copy(col_hbm.at[wid], cidx2)

        lane = lax.iota(jnp.int32, 16)

        def ebase(t):
            return pl.multiple_of(wid * EPW + t * CH, CH)

        def start_gather(t, b):
            pltpu.async_copy(nf_hbm.at[ridx2.at[t]], sbuf.at[b], gsem_r)
            pltpu.async_copy(nf_hbm.at[cidx2.at[t]], dbuf.at[b], gsem_c)
            pltpu.async_copy(ea_hbm.at[:, pl.ds(ebase(t), CH)], abuf.at[b],
                             gsem_a)

        def wait_gather(t, b):
            pltpu.make_async_copy(nf_hbm.at[ridx2.at[t]], sbuf.at[b], gsem_r).wait()
            pltpu.make_async_copy(nf_hbm.at[cidx2.at[t]], dbuf.at[b], gsem_c).wait()
            pltpu.make_async_copy(ea_hbm.at[:, pl.ds(ebase(t), CH)], abuf.at[b],
                                  gsem_a).wait()

        def merge_ea(b):
            for c in range(EDGE_NF):
                cols = jnp.full((16,), NODE_NF + c, jnp.int32)
                for v in range(CH // 16):
                    vec = abuf[b, c, pl.ds(16 * v, 16)]
                    plsc.store_scatter(sbuf.at[b], [lane + 16 * v, cols], vec)

        def out_slices(t):
            base = ebase(t)
            return (g_hbm.at[pl.ds(base, CH), pl.ds(0, 16)],
                    g_hbm.at[pl.ds(base, CH), pl.ds(16, 16)])

        def start_write(t, b):
            o_s, o_d = out_slices(t)
            pltpu.async_copy(sbuf.at[b], o_s, wsem)
            pltpu.async_copy(dbuf.at[b], o_d, wsem)

        def wait_write(t, b):
            o_s, o_d = out_slices(t)
            pltpu.make_async_copy(sbuf.at[b], o_s, wsem).wait()
            pltpu.make_async_copy(dbuf.at[b], o_d, wsem).wait()

        for t in range(GLEAD):
            start_gather(t, t % RG)

        def body(g, carry):
            for b_off in range(RG):
                t = g * RG + b_off
                b = b_off
                bw = (b_off - 2) % RG

                @pl.when(t >= 2)
                def _():
                    wait_write(t - 2, bw)

                @pl.when(t + GLEAD < NCH)
                def _():
                    start_gather(t + GLEAD, bw)

                wait_gather(t, b)
                merge_ea(b)
                start_write(t, b)
            return carry

        lax.fori_loop(0, NCH // RG, body, 0, unroll=False)
        for t in range(NCH - NCH % RG, NCH):
            b = t % RG
            bw = (b - 2) % RG
            wait_write(t - 2, bw)

            @pl.when(t + GLEAD < NCH)
            def _():
                start_gather(t + GLEAD, bw)

            wait_gather(t, b)
            merge_ea(b)
            start_write(t, b)
        wait_write(NCH - 2, (NCH - 2) % RG)
        wait_write(NCH - 1, (NCH - 1) % RG)

    return k(nf16, row3, col3, eaT)


RS = 2
SLEAD = 1


def _scatter_call(ef, row3, zeros_big):
    @functools.partial(
        pl.kernel,
        out_type=(
            jax.ShapeDtypeStruct((N_PAD, H_NF), _F32),
            jax.ShapeDtypeStruct((N_PAD, H_NF), _F32),
        ),
        mesh=_sc_mesh(),
        scratch_types=[
            pltpu.VMEM((NCH, CH), jnp.int32),
            pltpu.VMEM((RS, CH, H_NF), _F32),
            pltpu.VMEM_SHARED((N_PAD, H_NF), _F32),
            pltpu.SemaphoreType.DMA,
            pltpu.SemaphoreType.DMA,
        ],
        compiler_params=pltpu.CompilerParams(use_tc_tiling_on_sc=False),
    )
    def k(ef_hbm, row_hbm, z_hbm, p0_hbm, p1_hbm, idx2, ebuf, agg_sh,
          lsem, asem):
        c = lax.axis_index("c")
        s = lax.axis_index("s")
        wid = c * NS + s
        rslice = pl.ds(s * RPT, RPT)
        pltpu.sync_copy(z_hbm.at[rslice], agg_sh.at[rslice])
        pltpu.sync_copy(row_hbm.at[wid], idx2)
        plsc.subcore_barrier()

        def _ef_pairs(t, b):
            blk = wid * (EPW // BE) + t // (BE // CH)
            i0 = pl.multiple_of((t % (BE // CH)) * (CH // 4), CH // 4)
            return [(ef_hbm.at[blk, j, pl.ds(i0, CH // 4)],
                     ebuf.at[b, pl.ds(j * (CH // 4), CH // 4)])
                    for j in range(4)]

        def start_load(t, b):
            for src, dst in _ef_pairs(t, b):
                pltpu.async_copy(src, dst, lsem)

        def wait_load(t, b):
            for src, dst in _ef_pairs(t, b):
                pltpu.make_async_copy(src, dst, lsem).wait()

        def start_add(t, b):
            pltpu.async_copy(ebuf.at[b], agg_sh.at[idx2.at[t]], asem, add=True)

        def wait_add(t, b):
            pltpu.make_async_copy(ebuf.at[b], agg_sh.at[idx2.at[t]], asem).wait()

        start_load(0, 0)

        def body(g, carry):
            for b in range(RS):
                t = g * RS + b
                bo = 1 - b

                @pl.when(t >= 1)
                def _():
                    wait_add(t - 1, bo)

                @pl.when(t + 1 < NCH)
                def _():
                    start_load(t + 1, bo)

                wait_load(t, b)
                start_add(t, b)
            return carry

        lax.fori_loop(0, NCH // RS, body, 0, unroll=False)
        wait_add(NCH - 1, (NCH - 1) % RS)
        plsc.subcore_barrier()

        @pl.when(c == 0)
        def _():
            pltpu.sync_copy(agg_sh.at[rslice], p0_hbm.at[rslice])

        @pl.when(c == 1)
        def _():
            pltpu.sync_copy(agg_sh.at[rslice], p1_hbm.at[rslice])

    return k(ef, row3, zeros_big)


BE = 2048


_BF16 = jnp.bfloat16


BE4 = BE // 4
NB = E_PAD // BE


def _edge_mlp_kernel(x4, w1big, b1, w2, b2, out):
    xb = x4[...].astype(_BF16)
    for j in range(4):
        h = jnp.dot(xb, w1big[j], preferred_element_type=_F32)
        h = jnp.maximum(h + b1[...], 0.0)
        h = jnp.dot(h.astype(_BF16), w2[...],
                    preferred_element_type=_F32) + b2[...]
        out[0, j] = jnp.maximum(h, 0.0)


def _edge_mlp_call(x4, w1big, b1, w2, b2):
    bcast = lambda shape: pl.BlockSpec(shape, lambda i: tuple(0 for _ in shape))
    return pl.pallas_call(
        _edge_mlp_kernel,
        grid=(NB,),
        in_specs=[
            pl.BlockSpec((BE4, H_NF), lambda i: (i, 0)),
            bcast((4, H_NF, H_NF)),
            bcast((1, H_NF)),
            bcast((H_NF, H_NF)),
            bcast((1, H_NF)),
        ],
        out_specs=pl.BlockSpec((1, 4, BE4, H_NF), lambda i: (i, 0, 0, 0)),
        out_shape=jax.ShapeDtypeStruct((NB, 4, BE4, H_NF), _F32),
    )(x4, w1big, b1, w2, b2)


def _permute_idx_kernel(idx2d, perm, out):
    xf = idx2d[...].astype(_F32)
    pf = perm[...]
    out[...] = jnp.dot(xf, pf, preferred_element_type=_F32,
                       precision=lax.Precision.HIGHEST).astype(jnp.int32)


def _permute_idx_call(idx2d, perm):
    nrows = E_PAD // CH
    return pl.pallas_call(
        _permute_idx_kernel,
        grid=(2,),
        in_specs=[
            pl.BlockSpec((nrows // 2, CH), lambda i: (i, 0)),
            pl.BlockSpec((CH, CH), lambda i: (0, 0)),
        ],
        out_specs=pl.BlockSpec((nrows // 2, CH), lambda i: (i, 0)),
        out_shape=jax.ShapeDtypeStruct((nrows, CH), jnp.int32),
    )(idx2d, perm)


BN = 1024


def _node_mlp_kernel(nf, p0, p1, w1n, w1a, b1, w2, b2, fw, fb, out):
    agg = p0[...] + p1[...]
    h = jnp.dot(nf[...], w1n[...], preferred_element_type=_F32)
    h = h + jnp.dot(agg, w1a[...], preferred_element_type=_F32)
    h = jnp.maximum(h + b1[...], 0.0)
    h = jnp.dot(h, w2[...], preferred_element_type=_F32) + b2[...]
    out[...] = jnp.dot(h, fw[...], preferred_element_type=_F32) + fb[...]


def _node_mlp_call(nf16, p0, p1, w1n, w1a, b1, w2, b2, fw8, fb8):
    grid = (pl.cdiv(N_NODES, BN),)
    bcast = lambda shape: pl.BlockSpec(shape, lambda i: (0, 0))
    return pl.pallas_call(
        _node_mlp_kernel,
        grid=grid,
        in_specs=[
            pl.BlockSpec((BN, 16), lambda i: (i, 0)),
            pl.BlockSpec((BN, H_NF), lambda i: (i, 0)),
            pl.BlockSpec((BN, H_NF), lambda i: (i, 0)),
            bcast((16, H_NF)),
            bcast((H_NF, H_NF)),
            bcast((1, H_NF)),
            bcast((H_NF, H_NF)),
            bcast((1, H_NF)),
            bcast((H_NF, 8)),
            bcast((1, 8)),
        ],
        out_specs=pl.BlockSpec((BN, 8), lambda i: (i, 0)),
        out_shape=jax.ShapeDtypeStruct((N_NODES, 8), _F32),
    )(nf16, p0, p1, w1n, w1a, b1, w2, b2, fw8, fb8)


def kernel(node_feats, edge_index, edge_attr,
           eW1, eb1, eW2, eb2, nW1, nb1, nW2, nb2, fW, fb):
    row = edge_index[0]
    col = edge_index[1]
    pad_idx = jnp.full((E_PAD - N_EDGES,), N_NODES, jnp.int32)
    row_pad = jnp.concatenate([row, pad_idx])
    row3 = row_pad.reshape(NW, NCH, CH)
    col3 = jnp.concatenate([col, pad_idx]).reshape(NW, NCH, CH)
    src_of = 4 * (jnp.arange(CH) % (CH // 4)) + jnp.arange(CH) // (CH // 4)
    perm = jnp.zeros((CH, CH), _F32).at[src_of, jnp.arange(CH)].set(1.0)
    row3p = _permute_idx_call(row_pad.reshape(E_PAD // CH, CH),
                              perm).reshape(NW, NCH, CH)

    nf16 = jnp.zeros((N_PAD, 16), _F32).at[:N_NODES, :NODE_NF].set(node_feats)
    eaT = jnp.pad(edge_attr.T, ((0, 0), (0, E_PAD - N_EDGES)))

    w1sd = jnp.zeros((32, H_NF), _F32)
    w1sd = w1sd.at[:NODE_NF].set(eW1[:NODE_NF])
    w1sd = w1sd.at[NODE_NF:NODE_NF + EDGE_NF].set(eW1[2 * NODE_NF:])
    w1sd = w1sd.at[16:16 + NODE_NF].set(eW1[NODE_NF:2 * NODE_NF])
    w1big = jnp.zeros((4, H_NF, H_NF), _F32)
    for j in range(4):
        w1big = w1big.at[j, 32 * j:32 * j + 32].set(w1sd)
    e_b1 = eb1.reshape(1, H_NF)
    e_b2 = eb2.reshape(1, H_NF)

    w1n = jnp.zeros((16, H_NF), _F32).at[:NODE_NF].set(nW1[:NODE_NF])
    w1a = nW1[NODE_NF:]
    n_b1 = nb1.reshape(1, H_NF)
    n_b2 = nb2.reshape(1, H_NF)
    fw8 = jnp.zeros((H_NF, 8), _F32).at[:, :EMB_NF].set(fW)
    fb8 = jnp.zeros((1, 8), _F32).at[0, :EMB_NF].set(fb)

    g32 = _gather_call(nf16, row3, col3, eaT)
    x4 = g32.reshape(E_PAD // 4, H_NF)
    ef4 = _edge_mlp_call(x4, w1big.astype(_BF16),
                         e_b1, eW2.astype(_BF16), e_b2)
    zeros_big = jnp.zeros((N_PAD, H_NF), _F32)
    p0, p1 = _scatter_call(ef4, row3p, zeros_big)
    out8 = _node_mlp_call(nf16, p0, p1, w1n, w1a, n_b1, nW2, n_b2, fw8, fb8)
    return out8[:, :EMB_NF]


# speedup vs baseline: 1.9371x; 1.0014x over previous
import functools

import jax
import jax.numpy as jnp
from jax import lax
from jax.experimental import pallas as pl
from jax.experimental.pallas import tpu as pltpu
from jax.experimental.pallas import tpu_sc as plsc

N_NODES = 10000
N_EDGES = 320000
NODE_NF = 11
EDGE_NF = 4
H_NF = 128
EMB_NF = 4

NC = 2
NS = 16
NW = NC * NS

CH = 128
EPW = 10240
NCH = EPW // CH
E_PAD = EPW * NW
N_PAD = 10112
RPT = N_PAD // NS

_F32 = jnp.float32


def _sc_mesh():
    return plsc.VectorSubcoreMesh(
        core_axis_name="c", subcore_axis_name="s", num_cores=NC, num_subcores=NS
    )


RG = 12
GLEAD = 10


def _gather_call(nf16, row3, col3, eaT):
    @functools.partial(
        pl.kernel,
        out_type=jax.ShapeDtypeStruct((E_PAD, 32), _F32),
        mesh=_sc_mesh(),
        scratch_types=[
            pltpu.VMEM((NCH, CH), jnp.int32),
            pltpu.VMEM((NCH, CH), jnp.int32),
            pltpu.VMEM((RG, CH, 16), _F32),
            pltpu.VMEM((RG, CH, 16), _F32),
            pltpu.VMEM((RG, EDGE_NF, CH), _F32),
            pltpu.SemaphoreType.DMA,
            pltpu.SemaphoreType.DMA,
            pltpu.SemaphoreType.DMA,
            pltpu.SemaphoreType.DMA,
        ],
        compiler_params=pltpu.CompilerParams(use_tc_tiling_on_sc=False,
                                             needs_layout_passes=False),
    )
    def k(nf_hbm, row_hbm, col_hbm, ea_hbm, g_hbm,
          ridx2, cidx2, sbuf, dbuf, abuf, gsem_r, gsem_c, gsem_a, wsem):
        wid = lax.axis_index("c") * NS + lax.axis_index("s")

        pltpu.sync_copy(row_hbm.at[wid], ridx2)
        pltpu.sync_copy(col_hbm.at[wid], cidx2)

        lane = lax.iota(jnp.int32, 16)

        def ebase(t):
            return pl.multiple_of(wid * EPW + t * CH, CH)

        def start_gather(t, b):
            pltpu.async_copy(nf_hbm.at[ridx2.at[t]], sbuf.at[b], gsem_r)
            pltpu.async_copy(nf_hbm.at[cidx2.at[t]], dbuf.at[b], gsem_c)
            pltpu.async_copy(ea_hbm.at[:, pl.ds(ebase(t), CH)], abuf.at[b],
                             gsem_a)

        def wait_gather(t, b):
            pltpu.make_async_copy(nf_hbm.at[ridx2.at[t]], sbuf.at[b], gsem_r).wait()
            pltpu.make_async_copy(nf_hbm.at[cidx2.at[t]], dbuf.at[b], gsem_c).wait()
            pltpu.make_async_copy(ea_hbm.at[:, pl.ds(ebase(t), CH)], abuf.at[b],
                                  gsem_a).wait()

        def merge_ea(b):
            for c in range(EDGE_NF):
                cols = jnp.full((16,), NODE_NF + c, jnp.int32)
                for v in range(CH // 16):
                    vec = abuf[b, c, pl.ds(16 * v, 16)]
                    plsc.store_scatter(sbuf.at[b], [lane + 16 * v, cols], vec)

        def out_slices(t):
            base = ebase(t)
            return (g_hbm.at[pl.ds(base, CH), pl.ds(0, 16)],
                    g_hbm.at[pl.ds(base, CH), pl.ds(16, 16)])

        def start_write(t, b):
            o_s, o_d = out_slices(t)
            pltpu.async_copy(sbuf.at[b], o_s, wsem)
            pltpu.async_copy(dbuf.at[b], o_d, wsem)

        def wait_write(t, b):
            o_s, o_d = out_slices(t)
            pltpu.make_async_copy(sbuf.at[b], o_s, wsem).wait()
            pltpu.make_async_copy(dbuf.at[b], o_d, wsem).wait()

        for t in range(GLEAD):
            start_gather(t, t % RG)

        def body(g, carry):
            for b_off in range(RG):
                t = g * RG + b_off
                b = b_off
                bw = (b_off - 2) % RG

                @pl.when(t >= 2)
                def _():
                    wait_write(t - 2, bw)

                @pl.when(t + GLEAD < NCH)
                def _():
                    start_gather(t + GLEAD, bw)

                wait_gather(t, b)
                merge_ea(b)
                start_write(t, b)
            return carry

        lax.fori_loop(0, NCH // RG, body, 0, unroll=False)
        for t in range(NCH - NCH % RG, NCH):
            b = t % RG
            bw = (b - 2) % RG
            wait_write(t - 2, bw)

            @pl.when(t + GLEAD < NCH)
            def _():
                start_gather(t + GLEAD, bw)

            wait_gather(t, b)
            merge_ea(b)
            start_write(t, b)
        wait_write(NCH - 2, (NCH - 2) % RG)
        wait_write(NCH - 1, (NCH - 1) % RG)

    return k(nf16, row3, col3, eaT)


RS = 2
SLEAD = 1


def _scatter_call(ef, row3, zeros_big):
    @functools.partial(
        pl.kernel,
        out_type=(
            jax.ShapeDtypeStruct((N_PAD, H_NF), _F32),
            jax.ShapeDtypeStruct((N_PAD, H_NF), _F32),
        ),
        mesh=_sc_mesh(),
        scratch_types=[
            pltpu.VMEM((NCH, CH), jnp.int32),
            pltpu.VMEM((RS, CH, H_NF), _F32),
            pltpu.VMEM_SHARED((N_PAD, H_NF), _F32),
            pltpu.SemaphoreType.DMA,
            pltpu.SemaphoreType.DMA,
        ],
        compiler_params=pltpu.CompilerParams(use_tc_tiling_on_sc=False),
    )
    def k(ef_hbm, row_hbm, z_hbm, p0_hbm, p1_hbm, idx2, ebuf, agg_sh,
          lsem, asem):
        c = lax.axis_index("c")
        s = lax.axis_index("s")
        wid = c * NS + s
        rslice = pl.ds(s * RPT, RPT)
        pltpu.sync_copy(z_hbm.at[rslice], agg_sh.at[rslice])
        pltpu.sync_copy(row_hbm.at[wid], idx2)
        plsc.subcore_barrier()

        def _ef_pairs(t, b):
            blk = wid * (EPW // BE) + t // (BE // CH)
            i0 = pl.multiple_of((t % (BE // CH)) * (CH // 4), CH // 4)
            return [(ef_hbm.at[blk, j, pl.ds(i0, CH // 4)],
                     ebuf.at[b, pl.ds(j * (CH // 4), CH // 4)])
                    for j in range(4)]

        def start_load(t, b):
            for src, dst in _ef_pairs(t, b):
                pltpu.async_copy(src, dst, lsem)

        def wait_load(t, b):
            for src, dst in _ef_pairs(t, b):
                pltpu.make_async_copy(src, dst, lsem).wait()

        def start_add(t, b):
            pltpu.async_copy(ebuf.at[b], agg_sh.at[idx2.at[t]], asem, add=True)

        def wait_add(t, b):
            pltpu.make_async_copy(ebuf.at[b], agg_sh.at[idx2.at[t]], asem).wait()

        start_load(0, 0)

        def body(g, carry):
            for b in range(RS):
                t = g * RS + b
                bo = 1 - b

                @pl.when(t >= 1)
                def _():
                    wait_add(t - 1, bo)

                @pl.when(t + 1 < NCH)
                def _():
                    start_load(t + 1, bo)

                wait_load(t, b)
                start_add(t, b)
            return carry

        lax.fori_loop(0, NCH // RS, body, 0, unroll=False)
        wait_add(NCH - 1, (NCH - 1) % RS)
        plsc.subcore_barrier()

        @pl.when(c == 0)
        def _():
            pltpu.sync_copy(agg_sh.at[rslice], p0_hbm.at[rslice])

        @pl.when(c == 1)
        def _():
            pltpu.sync_copy(agg_sh.at[rslice], p1_hbm.at[rslice])

    return k(ef, row3, zeros_big)


BE = 2048


_BF16 = jnp.bfloat16


BE4 = BE // 4
NB = E_PAD // BE


def _edge_mlp_kernel(x4, w1big, b1, w2, b2, out):
    xb = x4[...].astype(_BF16)
    for j in range(4):
        h = jnp.dot(xb, w1big[j], preferred_element_type=_F32)
        h = jnp.maximum(h + b1[...], 0.0)
        h = jnp.dot(h.astype(_BF16), w2[...],
                    preferred_element_type=_F32) + b2[...]
        out[0, j] = jnp.maximum(h, 0.0)


def _edge_mlp_call(x4, w1big, b1, w2, b2):
    bcast = lambda shape: pl.BlockSpec(shape, lambda i: tuple(0 for _ in shape))
    return pl.pallas_call(
        _edge_mlp_kernel,
        grid=(NB,),
        in_specs=[
            pl.BlockSpec((BE4, H_NF), lambda i: (i, 0)),
            bcast((4, H_NF, H_NF)),
            bcast((1, H_NF)),
            bcast((H_NF, H_NF)),
            bcast((1, H_NF)),
        ],
        out_specs=pl.BlockSpec((1, 4, BE4, H_NF), lambda i: (i, 0, 0, 0)),
        out_shape=jax.ShapeDtypeStruct((NB, 4, BE4, H_NF), _F32),
    )(x4, w1big, b1, w2, b2)


def _permute_idx_kernel(idx2d, perm, out):
    xf = idx2d[...].astype(_F32)
    pf = perm[...]
    out[...] = jnp.dot(xf, pf, preferred_element_type=_F32,
                       precision=lax.Precision.HIGHEST).astype(jnp.int32)


def _permute_idx_call(idx2d, perm):
    nrows = E_PAD // CH
    return pl.pallas_call(
        _permute_idx_kernel,
        grid=(2,),
        in_specs=[
            pl.BlockSpec((nrows // 2, CH), lambda i: (i, 0)),
            pl.BlockSpec((CH, CH), lambda i: (0, 0)),
        ],
        out_specs=pl.BlockSpec((nrows // 2, CH), lambda i: (i, 0)),
        out_shape=jax.ShapeDtypeStruct((nrows, CH), jnp.int32),
    )(idx2d, perm)


BN = 1024


def _node_mlp_kernel(nf, p0, p1, w1n, w1a, b1, w2, b2, fw, fb, out):
    agg = p0[...] + p1[...]
    h = jnp.dot(nf[...], w1n[...], preferred_element_type=_F32)
    h = h + jnp.dot(agg, w1a[...], preferred_element_type=_F32)
    h = jnp.maximum(h + b1[...], 0.0)
    h = jnp.dot(h, w2[...], preferred_element_type=_F32) + b2[...]
    out[...] = jnp.dot(h, fw[...], preferred_element_type=_F32) + fb[...]


def _node_mlp_call(nf16, p0, p1, w1n, w1a, b1, w2, b2, fw8, fb8):
    grid = (pl.cdiv(N_NODES, BN),)
    bcast = lambda shape: pl.BlockSpec(shape, lambda i: (0, 0))
    return pl.pallas_call(
        _node_mlp_kernel,
        grid=grid,
        in_specs=[
            pl.BlockSpec((BN, 16), lambda i: (i, 0)),
            pl.BlockSpec((BN, H_NF), lambda i: (i, 0)),
            pl.BlockSpec((BN, H_NF), lambda i: (i, 0)),
            bcast((16, H_NF)),
            bcast((H_NF, H_NF)),
            bcast((1, H_NF)),
            bcast((H_NF, H_NF)),
            bcast((1, H_NF)),
            bcast((H_NF, 8)),
            bcast((1, 8)),
        ],
        out_specs=pl.BlockSpec((BN, 8), lambda i: (i, 0)),
        out_shape=jax.ShapeDtypeStruct((N_NODES, 8), _F32),
    )(nf16, p0, p1, w1n, w1a, b1, w2, b2, fw8, fb8)


def kernel(node_feats, edge_index, edge_attr,
           eW1, eb1, eW2, eb2, nW1, nb1, nW2, nb2, fW, fb):
    row = edge_index[0]
    col = edge_index[1]
    pad_idx = jnp.full((E_PAD - N_EDGES,), N_NODES, jnp.int32)
    row_pad = jnp.concatenate([row, pad_idx])
    row3 = row_pad.reshape(NW, NCH, CH)
    col3 = jnp.concatenate([col, pad_idx]).reshape(NW, NCH, CH)
    src_of = 4 * (jnp.arange(CH) % (CH // 4)) + jnp.arange(CH) // (CH // 4)
    perm = jnp.zeros((CH, CH), _F32).at[src_of, jnp.arange(CH)].set(1.0)
    row3p = _permute_idx_call(row_pad.reshape(E_PAD // CH, CH),
                              perm).reshape(NW, NCH, CH)

    nf16 = jnp.zeros((N_PAD, 16), _F32).at[:N_NODES, :NODE_NF].set(node_feats)
    eaT = jnp.pad(edge_attr.T, ((0, 0), (0, E_PAD - N_EDGES)))

    w1sd = jnp.zeros((32, H_NF), _F32)
    w1sd = w1sd.at[:NODE_NF].set(eW1[:NODE_NF])
    w1sd = w1sd.at[NODE_NF:NODE_NF + EDGE_NF].set(eW1[2 * NODE_NF:])
    w1sd = w1sd.at[16:16 + NODE_NF].set(eW1[NODE_NF:2 * NODE_NF])
    w1big = jnp.zeros((4, H_NF, H_NF), _F32)
    for j in range(4):
        w1big = w1big.at[j, 32 * j:32 * j + 32].set(w1sd)
    e_b1 = eb1.reshape(1, H_NF)
    e_b2 = eb2.reshape(1, H_NF)

    w1n = jnp.zeros((16, H_NF), _F32).at[:NODE_NF].set(nW1[:NODE_NF])
    w1a = nW1[NODE_NF:]
    n_b1 = nb1.reshape(1, H_NF)
    n_b2 = nb2.reshape(1, H_NF)
    fw8 = jnp.zeros((H_NF, 8), _F32).at[:, :EMB_NF].set(fW)
    fb8 = jnp.zeros((1, 8), _F32).at[0, :EMB_NF].set(fb)

    g32 = _gather_call(nf16, row3, col3, eaT)
    x4 = g32.reshape(E_PAD // 4, H_NF)
    ef4 = _edge_mlp_call(x4, w1big.astype(_BF16),
                         e_b1, eW2.astype(_BF16), e_b2)
    zeros_big = jnp.zeros((N_PAD, H_NF), _F32)
    p0, p1 = _scatter_call(ef4, row3p, zeros_big)
    out8 = _node_mlp_call(nf16, p0, p1, w1n, w1a, n_b1, nW2, n_b2, fw8, fb8)
    return out8[:, :EMB_NF]


# speedup vs baseline: 2.4949x; 1.2880x over previous
import functools

import jax
import jax.numpy as jnp
from jax import lax
from jax.experimental import pallas as pl
from jax.experimental.pallas import tpu as pltpu
from jax.experimental.pallas import tpu_sc as plsc

N_NODES = 10000
N_EDGES = 320000
NODE_NF = 11
EDGE_NF = 4
H_NF = 128
EMB_NF = 4

NC = 2
NS = 16
NW = NC * NS

CH = 128
EPW = 10240
NCH = EPW // CH
E_PAD = EPW * NW
N_PAD = 10112
RPT = N_PAD // NS

_F32 = jnp.float32


def _sc_mesh():
    return plsc.VectorSubcoreMesh(
        core_axis_name="c", subcore_axis_name="s", num_cores=NC, num_subcores=NS
    )


RG = 12
GLEAD = 10


def _gather_call(nf16, row3, col3, eaT):
    @functools.partial(
        pl.kernel,
        out_type=jax.ShapeDtypeStruct((E_PAD, 32), _F32),
        mesh=_sc_mesh(),
        scratch_types=[
            pltpu.VMEM((NCH, CH), jnp.int32),
            pltpu.VMEM((NCH, CH), jnp.int32),
            pltpu.VMEM((RG, CH, 16), _F32),
            pltpu.VMEM((RG, CH, 16), _F32),
            pltpu.VMEM((RG, EDGE_NF, CH), _F32),
            pltpu.VMEM_SHARED((N_PAD, 16), _F32),
            pltpu.SemaphoreType.DMA,
            pltpu.SemaphoreType.DMA,
            pltpu.SemaphoreType.DMA,
            pltpu.SemaphoreType.DMA,
        ],
        compiler_params=pltpu.CompilerParams(use_tc_tiling_on_sc=False,
                                             needs_layout_passes=False),
    )
    def k(nf_hbm, row_hbm, col_hbm, ea_hbm, g_hbm,
          ridx2, cidx2, sbuf, dbuf, abuf, nf_sp, gsem_r, gsem_c, gsem_a, wsem):
        s = lax.axis_index("s")
        wid = lax.axis_index("c") * NS + s

        tslice = pl.ds(s * RPT, RPT)
        pltpu.sync_copy(nf_hbm.at[tslice], nf_sp.at[tslice])

        pltpu.sync_copy(row_hbm.at[wid], ridx2)
        pltpu.sync_copy(col_hbm.at[wid], cidx2)
        plsc.subcore_barrier()

        lane = lax.iota(jnp.int32, 16)

        def ebase(t):
            return pl.multiple_of(wid * EPW + t * CH, CH)

        def start_gather(t, b):
            pltpu.async_copy(nf_sp.at[ridx2.at[t]], sbuf.at[b], gsem_r)
            pltpu.async_copy(nf_sp.at[cidx2.at[t]], dbuf.at[b], gsem_c)
            pltpu.async_copy(ea_hbm.at[:, pl.ds(ebase(t), CH)], abuf.at[b],
                             gsem_a)

        def wait_gather(t, b):
            pltpu.make_async_copy(nf_sp.at[ridx2.at[t]], sbuf.at[b], gsem_r).wait()
            pltpu.make_async_copy(nf_sp.at[cidx2.at[t]], dbuf.at[b], gsem_c).wait()
            pltpu.make_async_copy(ea_hbm.at[:, pl.ds(ebase(t), CH)], abuf.at[b],
                                  gsem_a).wait()

        def merge_ea(b):
            for c in range(EDGE_NF):
                cols = jnp.full((16,), NODE_NF + c, jnp.int32)
                for v in range(CH // 16):
                    vec = abuf[b, c, pl.ds(16 * v, 16)]
                    plsc.store_scatter(sbuf.at[b], [lane + 16 * v, cols], vec)

        def out_slices(t):
            base = ebase(t)
            return (g_hbm.at[pl.ds(base, CH), pl.ds(0, 16)],
                    g_hbm.at[pl.ds(base, CH), pl.ds(16, 16)])

        def start_write(t, b):
            o_s, o_d = out_slices(t)
            pltpu.async_copy(sbuf.at[b], o_s, wsem)
            pltpu.async_copy(dbuf.at[b], o_d, wsem)

        def wait_write(t, b):
            o_s, o_d = out_slices(t)
            pltpu.make_async_copy(sbuf.at[b], o_s, wsem).wait()
            pltpu.make_async_copy(dbuf.at[b], o_d, wsem).wait()

        for t in range(GLEAD):
            start_gather(t, t % RG)

        def body(g, carry):
            for b_off in range(RG):
                t = g * RG + b_off
                b = b_off
                bw = (b_off - 2) % RG

                @pl.when(t >= 2)
                def _():
                    wait_write(t - 2, bw)

                @pl.when(t + GLEAD < NCH)
                def _():
                    start_gather(t + GLEAD, bw)

                wait_gather(t, b)
                merge_ea(b)
                start_write(t, b)
            return carry

        lax.fori_loop(0, NCH // RG, body, 0, unroll=False)
        for t in range(NCH - NCH % RG, NCH):
            b = t % RG
            bw = (b - 2) % RG
            wait_write(t - 2, bw)

            @pl.when(t + GLEAD < NCH)
            def _():
                start_gather(t + GLEAD, bw)

            wait_gather(t, b)
            merge_ea(b)
            start_write(t, b)
        wait_write(NCH - 2, (NCH - 2) % RG)
        wait_write(NCH - 1, (NCH - 1) % RG)

    return k(nf16, row3, col3, eaT)


RS = 2
SLEAD = 1


def _scatter_call(ef, row3, zeros_big):
    @functools.partial(
        pl.kernel,
        out_type=(
            jax.ShapeDtypeStruct((N_PAD, H_NF), _F32),
            jax.ShapeDtypeStruct((N_PAD, H_NF), _F32),
        ),
        mesh=_sc_mesh(),
        scratch_types=[
            pltpu.VMEM((NCH, CH), jnp.int32),
            pltpu.VMEM((RS, CH, H_NF), _F32),
            pltpu.VMEM_SHARED((N_PAD, H_NF), _F32),
            pltpu.SemaphoreType.DMA,
            pltpu.SemaphoreType.DMA,
        ],
        compiler_params=pltpu.CompilerParams(use_tc_tiling_on_sc=False),
    )
    def k(ef_hbm, row_hbm, z_hbm, p0_hbm, p1_hbm, idx2, ebuf, agg_sh,
          lsem, asem):
        c = lax.axis_index("c")
        s = lax.axis_index("s")
        wid = c * NS + s
        rslice = pl.ds(s * RPT, RPT)
        pltpu.sync_copy(z_hbm.at[rslice], agg_sh.at[rslice])
        pltpu.sync_copy(row_hbm.at[wid], idx2)
        plsc.subcore_barrier()

        def _ef_pairs(t, b):
            blk = wid * (EPW // BE) + t // (BE // CH)
            i0 = pl.multiple_of((t % (BE // CH)) * (CH // 4), CH // 4)
            return [(ef_hbm.at[blk, j, pl.ds(i0, CH // 4)],
                     ebuf.at[b, pl.ds(j * (CH // 4), CH // 4)])
                    for j in range(4)]

        def start_load(t, b):
            for src, dst in _ef_pairs(t, b):
                pltpu.async_copy(src, dst, lsem)

        def wait_load(t, b):
            for src, dst in _ef_pairs(t, b):
                pltpu.make_async_copy(src, dst, lsem).wait()

        def start_add(t, b):
            pltpu.async_copy(ebuf.at[b], agg_sh.at[idx2.at[t]], asem, add=True)

        def wait_add(t, b):
            pltpu.make_async_copy(ebuf.at[b], agg_sh.at[idx2.at[t]], asem).wait()

        start_load(0, 0)

        def body(g, carry):
            for b in range(RS):
                t = g * RS + b
                bo = 1 - b

                @pl.when(t >= 1)
                def _():
                    wait_add(t - 1, bo)

                @pl.when(t + 1 < NCH)
                def _():
                    start_load(t + 1, bo)

                wait_load(t, b)
                start_add(t, b)
            return carry

        lax.fori_loop(0, NCH // RS, body, 0, unroll=False)
        wait_add(NCH - 1, (NCH - 1) % RS)
        plsc.subcore_barrier()

        @pl.when(c == 0)
        def _():
            pltpu.sync_copy(agg_sh.at[rslice], p0_hbm.at[rslice])

        @pl.when(c == 1)
        def _():
            pltpu.sync_copy(agg_sh.at[rslice], p1_hbm.at[rslice])

    return k(ef, row3, zeros_big)


BE = 2048


_BF16 = jnp.bfloat16


BE4 = BE // 4
NB = E_PAD // BE


def _edge_mlp_kernel(x4, w1big, b1, w2, b2, out):
    xb = x4[...].astype(_BF16)
    for j in range(4):
        h = jnp.dot(xb, w1big[j], preferred_element_type=_F32)
        h = jnp.maximum(h + b1[...], 0.0)
        h = jnp.dot(h.astype(_BF16), w2[...],
                    preferred_element_type=_F32) + b2[...]
        out[0, j] = jnp.maximum(h, 0.0)


def _edge_mlp_call(x4, w1big, b1, w2, b2):
    bcast = lambda shape: pl.BlockSpec(shape, lambda i: tuple(0 for _ in shape))
    return pl.pallas_call(
        _edge_mlp_kernel,
        grid=(NB,),
        in_specs=[
            pl.BlockSpec((BE4, H_NF), lambda i: (i, 0)),
            bcast((4, H_NF, H_NF)),
            bcast((1, H_NF)),
            bcast((H_NF, H_NF)),
            bcast((1, H_NF)),
        ],
        out_specs=pl.BlockSpec((1, 4, BE4, H_NF), lambda i: (i, 0, 0, 0)),
        out_shape=jax.ShapeDtypeStruct((NB, 4, BE4, H_NF), _F32),
    )(x4, w1big, b1, w2, b2)


def _permute_idx_kernel(idx2d, perm, out):
    xf = idx2d[...].astype(_F32)
    pf = perm[...]
    out[...] = jnp.dot(xf, pf, preferred_element_type=_F32,
                       precision=lax.Precision.HIGHEST).astype(jnp.int32)


def _permute_idx_call(idx2d, perm):
    nrows = E_PAD // CH
    return pl.pallas_call(
        _permute_idx_kernel,
        grid=(2,),
        in_specs=[
            pl.BlockSpec((nrows // 2, CH), lambda i: (i, 0)),
            pl.BlockSpec((CH, CH), lambda i: (0, 0)),
        ],
        out_specs=pl.BlockSpec((nrows // 2, CH), lambda i: (i, 0)),
        out_shape=jax.ShapeDtypeStruct((nrows, CH), jnp.int32),
    )(idx2d, perm)


BN = 1024


def _node_mlp_kernel(nf, p0, p1, w1n, w1a, b1, w2, b2, fw, fb, out):
    agg = p0[...] + p1[...]
    h = jnp.dot(nf[...], w1n[...], preferred_element_type=_F32)
    h = h + jnp.dot(agg, w1a[...], preferred_element_type=_F32)
    h = jnp.maximum(h + b1[...], 0.0)
    h = jnp.dot(h, w2[...], preferred_element_type=_F32) + b2[...]
    out[...] = jnp.dot(h, fw[...], preferred_element_type=_F32) + fb[...]


def _node_mlp_call(nf16, p0, p1, w1n, w1a, b1, w2, b2, fw8, fb8):
    grid = (pl.cdiv(N_NODES, BN),)
    bcast = lambda shape: pl.BlockSpec(shape, lambda i: (0, 0))
    return pl.pallas_call(
        _node_mlp_kernel,
        grid=grid,
        in_specs=[
            pl.BlockSpec((BN, 16), lambda i: (i, 0)),
            pl.BlockSpec((BN, H_NF), lambda i: (i, 0)),
            pl.BlockSpec((BN, H_NF), lambda i: (i, 0)),
            bcast((16, H_NF)),
            bcast((H_NF, H_NF)),
            bcast((1, H_NF)),
            bcast((H_NF, H_NF)),
            bcast((1, H_NF)),
            bcast((H_NF, 8)),
            bcast((1, 8)),
        ],
        out_specs=pl.BlockSpec((BN, 8), lambda i: (i, 0)),
        out_shape=jax.ShapeDtypeStruct((N_NODES, 8), _F32),
    )(nf16, p0, p1, w1n, w1a, b1, w2, b2, fw8, fb8)


def kernel(node_feats, edge_index, edge_attr,
           eW1, eb1, eW2, eb2, nW1, nb1, nW2, nb2, fW, fb):
    row = edge_index[0]
    col = edge_index[1]
    pad_idx = jnp.full((E_PAD - N_EDGES,), N_NODES, jnp.int32)
    row_pad = jnp.concatenate([row, pad_idx])
    row3 = row_pad.reshape(NW, NCH, CH)
    col3 = jnp.concatenate([col, pad_idx]).reshape(NW, NCH, CH)
    src_of = 4 * (jnp.arange(CH) % (CH // 4)) + jnp.arange(CH) // (CH // 4)
    perm = jnp.zeros((CH, CH), _F32).at[src_of, jnp.arange(CH)].set(1.0)
    row3p = _permute_idx_call(row_pad.reshape(E_PAD // CH, CH),
                              perm).reshape(NW, NCH, CH)

    nf16 = jnp.zeros((N_PAD, 16), _F32).at[:N_NODES, :NODE_NF].set(node_feats)
    eaT = jnp.pad(edge_attr.T, ((0, 0), (0, E_PAD - N_EDGES)))

    w1sd = jnp.zeros((32, H_NF), _F32)
    w1sd = w1sd.at[:NODE_NF].set(eW1[:NODE_NF])
    w1sd = w1sd.at[NODE_NF:NODE_NF + EDGE_NF].set(eW1[2 * NODE_NF:])
    w1sd = w1sd.at[16:16 + NODE_NF].set(eW1[NODE_NF:2 * NODE_NF])
    w1big = jnp.zeros((4, H_NF, H_NF), _F32)
    for j in range(4):
        w1big = w1big.at[j, 32 * j:32 * j + 32].set(w1sd)
    e_b1 = eb1.reshape(1, H_NF)
    e_b2 = eb2.reshape(1, H_NF)

    w1n = jnp.zeros((16, H_NF), _F32).at[:NODE_NF].set(nW1[:NODE_NF])
    w1a = nW1[NODE_NF:]
    n_b1 = nb1.reshape(1, H_NF)
    n_b2 = nb2.reshape(1, H_NF)
    fw8 = jnp.zeros((H_NF, 8), _F32).at[:, :EMB_NF].set(fW)
    fb8 = jnp.zeros((1, 8), _F32).at[0, :EMB_NF].set(fb)

    g32 = _gather_call(nf16, row3, col3, eaT)
    x4 = g32.reshape(E_PAD // 4, H_NF)
    ef4 = _edge_mlp_call(x4, w1big.astype(_BF16),
                         e_b1, eW2.astype(_BF16), e_b2)
    zeros_big = jnp.zeros((N_PAD, H_NF), _F32)
    p0, p1 = _scatter_call(ef4, row3p, zeros_big)
    out8 = _node_mlp_call(nf16, p0, p1, w1n, w1a, n_b1, nW2, n_b2, fw8, fb8)
    return out8[:, :EMB_NF]


# speedup vs baseline: 2.7177x; 1.0893x over previous
import functools

import jax
import jax.numpy as jnp
from jax import lax
from jax.experimental import pallas as pl
from jax.experimental.pallas import tpu as pltpu
from jax.experimental.pallas import tpu_sc as plsc

N_NODES = 10000
N_EDGES = 320000
NODE_NF = 11
EDGE_NF = 4
H_NF = 128
EMB_NF = 4

NC = 2
NS = 16
NW = NC * NS

CH = 128
EPW = 10240
NCH = EPW // CH
E_PAD = EPW * NW
N_PAD = 10112
RPT = N_PAD // NS

_F32 = jnp.float32


def _sc_mesh():
    return plsc.VectorSubcoreMesh(
        core_axis_name="c", subcore_axis_name="s", num_cores=NC, num_subcores=NS
    )


RG = 12
GLEAD = 10


def _gather_call(nf16, row3, col3, eaT):
    @functools.partial(
        pl.kernel,
        out_type=jax.ShapeDtypeStruct((E_PAD, 32), _F32),
        mesh=_sc_mesh(),
        scratch_types=[
            pltpu.VMEM((NCH, CH), jnp.int32),
            pltpu.VMEM((NCH, CH), jnp.int32),
            pltpu.VMEM((RG, CH, 16), _F32),
            pltpu.VMEM((RG, CH, 16), _F32),
            pltpu.VMEM((RG, EDGE_NF, CH), _F32),
            pltpu.VMEM_SHARED((N_PAD, 16), _F32),
            pltpu.SemaphoreType.DMA,
            pltpu.SemaphoreType.DMA,
            pltpu.SemaphoreType.DMA,
            pltpu.SemaphoreType.DMA,
        ],
        compiler_params=pltpu.CompilerParams(use_tc_tiling_on_sc=False,
                                             needs_layout_passes=False),
    )
    def k(nf_hbm, row_hbm, col_hbm, ea_hbm, g_hbm,
          ridx2, cidx2, sbuf, dbuf, abuf, nf_sp, gsem_r, gsem_c, gsem_a, wsem):
        s = lax.axis_index("s")
        wid = lax.axis_index("c") * NS + s

        tslice = pl.ds(s * RPT, RPT)
        pltpu.sync_copy(nf_hbm.at[tslice], nf_sp.at[tslice])

        pltpu.sync_copy(row_hbm.at[wid], ridx2)
        pltpu.sync_copy(col_hbm.at[wid], cidx2)
        plsc.subcore_barrier()

        lane = lax.iota(jnp.int32, 16)

        def ebase(t):
            return pl.multiple_of(wid * EPW + t * CH, CH)

        def start_gather(t, b):
            pltpu.async_copy(nf_sp.at[ridx2.at[t]], sbuf.at[b], gsem_r)
            pltpu.async_copy(nf_sp.at[cidx2.at[t]], dbuf.at[b], gsem_c)
            pltpu.async_copy(ea_hbm.at[:, pl.ds(ebase(t), CH)], abuf.at[b],
                             gsem_a)

        def wait_gather(t, b):
            pltpu.make_async_copy(nf_sp.at[ridx2.at[t]], sbuf.at[b], gsem_r).wait()
            pltpu.make_async_copy(nf_sp.at[cidx2.at[t]], dbuf.at[b], gsem_c).wait()
            pltpu.make_async_copy(ea_hbm.at[:, pl.ds(ebase(t), CH)], abuf.at[b],
                                  gsem_a).wait()

        def merge_ea(b):
            for c in range(EDGE_NF):
                cols = jnp.full((16,), NODE_NF + c, jnp.int32)
                for v in range(CH // 16):
                    vec = abuf[b, c, pl.ds(16 * v, 16)]
                    plsc.store_scatter(sbuf.at[b], [lane + 16 * v, cols], vec)

        def out_slices(t):
            base = ebase(t)
            return (g_hbm.at[pl.ds(base, CH), pl.ds(0, 16)],
                    g_hbm.at[pl.ds(base, CH), pl.ds(16, 16)])

        def start_write(t, b):
            o_s, o_d = out_slices(t)
            pltpu.async_copy(sbuf.at[b], o_s, wsem)
            pltpu.async_copy(dbuf.at[b], o_d, wsem)

        def wait_write(t, b):
            o_s, o_d = out_slices(t)
            pltpu.make_async_copy(sbuf.at[b], o_s, wsem).wait()
            pltpu.make_async_copy(dbuf.at[b], o_d, wsem).wait()

        for t in range(GLEAD):
            start_gather(t, t % RG)

        def body(g, carry):
            for b_off in range(RG):
                t = g * RG + b_off
                b = b_off
                bw = (b_off - 2) % RG

                @pl.when(t >= 2)
                def _():
                    wait_write(t - 2, bw)

                @pl.when(t + GLEAD < NCH)
                def _():
                    start_gather(t + GLEAD, bw)

                wait_gather(t, b)
                merge_ea(b)
                start_write(t, b)
            return carry

        lax.fori_loop(0, NCH // RG, body, 0, unroll=False)
        for t in range(NCH - NCH % RG, NCH):
            b = t % RG
            bw = (b - 2) % RG
            wait_write(t - 2, bw)

            @pl.when(t + GLEAD < NCH)
            def _():
                start_gather(t + GLEAD, bw)

            wait_gather(t, b)
            merge_ea(b)
            start_write(t, b)
        wait_write(NCH - 2, (NCH - 2) % RG)
        wait_write(NCH - 1, (NCH - 1) % RG)

    return k(nf16, row3, col3, eaT)


RS = 2
SLEAD = 1


def _scatter_call(ef, row3, zeros_big, epw, nch):
    @functools.partial(
        pl.kernel,
        out_type=(
            jax.ShapeDtypeStruct((N_PAD, H_NF), _F32),
            jax.ShapeDtypeStruct((N_PAD, H_NF), _F32),
        ),
        mesh=_sc_mesh(),
        scratch_types=[
            pltpu.VMEM((nch, CH), jnp.int32),
            pltpu.VMEM((RS, CH, H_NF), _F32),
            pltpu.VMEM_SHARED((N_PAD, H_NF), _F32),
            pltpu.SemaphoreType.DMA,
            pltpu.SemaphoreType.DMA,
        ],
        compiler_params=pltpu.CompilerParams(use_tc_tiling_on_sc=False),
    )
    def k(ef_hbm, row_hbm, z_hbm, p0_hbm, p1_hbm, idx2, ebuf, agg_sh,
          lsem, asem):
        c = lax.axis_index("c")
        s = lax.axis_index("s")
        wid = c * NS + s
        rslice = pl.ds(s * RPT, RPT)
        pltpu.sync_copy(z_hbm.at[rslice], agg_sh.at[rslice])
        pltpu.sync_copy(row_hbm.at[wid], idx2)
        plsc.subcore_barrier()

        def _ef_pairs(t, b):
            base = wid * epw + t * CH
            blk = base // BE
            i0 = pl.multiple_of((base % BE) // 4, CH // 4)
            return [(ef_hbm.at[blk, j, pl.ds(i0, CH // 4)],
                     ebuf.at[b, pl.ds(j * (CH // 4), CH // 4)])
                    for j in range(4)]

        def start_load(t, b):
            for src, dst in _ef_pairs(t, b):
                pltpu.async_copy(src, dst, lsem)

        def wait_load(t, b):
            for src, dst in _ef_pairs(t, b):
                pltpu.make_async_copy(src, dst, lsem).wait()

        def start_add(t, b):
            pltpu.async_copy(ebuf.at[b], agg_sh.at[idx2.at[t]], asem, add=True)

        def wait_add(t, b):
            pltpu.make_async_copy(ebuf.at[b], agg_sh.at[idx2.at[t]], asem).wait()

        start_load(0, 0)

        def body(g, carry):
            for b in range(RS):
                t = g * RS + b
                bo = 1 - b

                @pl.when(t >= 1)
                def _():
                    wait_add(t - 1, bo)

                @pl.when(t + 1 < nch)
                def _():
                    start_load(t + 1, bo)

                wait_load(t, b)
                start_add(t, b)
            return carry

        lax.fori_loop(0, nch // RS, body, 0, unroll=False)
        wait_add(nch - 1, (nch - 1) % RS)
        plsc.subcore_barrier()

        @pl.when(c == 0)
        def _():
            pltpu.sync_copy(agg_sh.at[rslice], p0_hbm.at[rslice])

        @pl.when(c == 1)
        def _():
            pltpu.sync_copy(agg_sh.at[rslice], p1_hbm.at[rslice])

    return k(ef, row3, zeros_big)


BE = 2048


_BF16 = jnp.bfloat16


BE4 = BE // 4
NB = E_PAD // BE


def _edge_mlp_kernel(x4, w1big, b1, w2, b2, out):
    xb = x4[...].astype(_BF16)
    for j in range(4):
        h = jnp.dot(xb, w1big[j], preferred_element_type=_F32)
        h = jnp.maximum(h + b1[...], 0.0)
        h = jnp.dot(h.astype(_BF16), w2[...],
                    preferred_element_type=_F32) + b2[...]
        out[0, j] = jnp.maximum(h, 0.0)


def _edge_mlp_call(x4, w1big, b1, w2, b2, blk0, nb):
    bcast = lambda shape: pl.BlockSpec(shape, lambda i: tuple(0 for _ in shape))
    return pl.pallas_call(
        _edge_mlp_kernel,
        grid=(nb,),
        in_specs=[
            pl.BlockSpec((BE4, H_NF), lambda i: (i + blk0, 0)),
            bcast((4, H_NF, H_NF)),
            bcast((1, H_NF)),
            bcast((H_NF, H_NF)),
            bcast((1, H_NF)),
        ],
        out_specs=pl.BlockSpec((1, 4, BE4, H_NF), lambda i: (i, 0, 0, 0)),
        out_shape=jax.ShapeDtypeStruct((nb, 4, BE4, H_NF), _F32),
    )(x4, w1big, b1, w2, b2)


def _permute_idx_kernel(idx2d, perm, out):
    xf = idx2d[...].astype(_F32)
    pf = perm[...]
    out[...] = jnp.dot(xf, pf, preferred_element_type=_F32,
                       precision=lax.Precision.HIGHEST).astype(jnp.int32)


def _permute_idx_call(idx2d, perm):
    nrows = E_PAD // CH
    return pl.pallas_call(
        _permute_idx_kernel,
        grid=(2,),
        in_specs=[
            pl.BlockSpec((nrows // 2, CH), lambda i: (i, 0)),
            pl.BlockSpec((CH, CH), lambda i: (0, 0)),
        ],
        out_specs=pl.BlockSpec((nrows // 2, CH), lambda i: (i, 0)),
        out_shape=jax.ShapeDtypeStruct((nrows, CH), jnp.int32),
    )(idx2d, perm)


BN = 1024


def _node_mlp_kernel(nf, p0, p1, p2, p3, w1n, w1a, b1, w2, b2, fw, fb, out):
    agg = (p0[...] + p1[...]) + (p2[...] + p3[...])
    h = jnp.dot(nf[...], w1n[...], preferred_element_type=_F32)
    h = h + jnp.dot(agg, w1a[...], preferred_element_type=_F32)
    h = jnp.maximum(h + b1[...], 0.0)
    h = jnp.dot(h, w2[...], preferred_element_type=_F32) + b2[...]
    out[...] = jnp.dot(h, fw[...], preferred_element_type=_F32) + fb[...]


def _node_mlp_call(nf16, p0, p1, p2, p3, w1n, w1a, b1, w2, b2, fw8, fb8):
    grid = (pl.cdiv(N_NODES, BN),)
    bcast = lambda shape: pl.BlockSpec(shape, lambda i: (0, 0))
    return pl.pallas_call(
        _node_mlp_kernel,
        grid=grid,
        in_specs=[
            pl.BlockSpec((BN, 16), lambda i: (i, 0)),
            pl.BlockSpec((BN, H_NF), lambda i: (i, 0)),
            pl.BlockSpec((BN, H_NF), lambda i: (i, 0)),
            pl.BlockSpec((BN, H_NF), lambda i: (i, 0)),
            pl.BlockSpec((BN, H_NF), lambda i: (i, 0)),
            bcast((16, H_NF)),
            bcast((H_NF, H_NF)),
            bcast((1, H_NF)),
            bcast((H_NF, H_NF)),
            bcast((1, H_NF)),
            bcast((H_NF, 8)),
            bcast((1, 8)),
        ],
        out_specs=pl.BlockSpec((BN, 8), lambda i: (i, 0)),
        out_shape=jax.ShapeDtypeStruct((N_NODES, 8), _F32),
    )(nf16, p0, p1, p2, p3, w1n, w1a, b1, w2, b2, fw8, fb8)


def kernel(node_feats, edge_index, edge_attr,
           eW1, eb1, eW2, eb2, nW1, nb1, nW2, nb2, fW, fb):
    row = edge_index[0]
    col = edge_index[1]
    pad_idx = jnp.full((E_PAD - N_EDGES,), N_NODES, jnp.int32)
    row_pad = jnp.concatenate([row, pad_idx])
    row3 = row_pad.reshape(NW, NCH, CH)
    col3 = jnp.concatenate([col, pad_idx]).reshape(NW, NCH, CH)
    src_of = 4 * (jnp.arange(CH) % (CH // 4)) + jnp.arange(CH) // (CH // 4)
    perm = jnp.zeros((CH, CH), _F32).at[src_of, jnp.arange(CH)].set(1.0)
    row4p = _permute_idx_call(row_pad.reshape(E_PAD // CH, CH),
                              perm).reshape(2, NW, NCH // 2, CH)

    nf16 = jnp.zeros((N_PAD, 16), _F32).at[:N_NODES, :NODE_NF].set(node_feats)
    eaT = jnp.pad(edge_attr.T, ((0, 0), (0, E_PAD - N_EDGES)))

    w1sd = jnp.zeros((32, H_NF), _F32)
    w1sd = w1sd.at[:NODE_NF].set(eW1[:NODE_NF])
    w1sd = w1sd.at[NODE_NF:NODE_NF + EDGE_NF].set(eW1[2 * NODE_NF:])
    w1sd = w1sd.at[16:16 + NODE_NF].set(eW1[NODE_NF:2 * NODE_NF])
    w1big = jnp.zeros((4, H_NF, H_NF), _F32)
    for j in range(4):
        w1big = w1big.at[j, 32 * j:32 * j + 32].set(w1sd)
    e_b1 = eb1.reshape(1, H_NF)
    e_b2 = eb2.reshape(1, H_NF)

    w1n = jnp.zeros((16, H_NF), _F32).at[:NODE_NF].set(nW1[:NODE_NF])
    w1a = nW1[NODE_NF:]
    n_b1 = nb1.reshape(1, H_NF)
    n_b2 = nb2.reshape(1, H_NF)
    fw8 = jnp.zeros((H_NF, 8), _F32).at[:, :EMB_NF].set(fW)
    fb8 = jnp.zeros((1, 8), _F32).at[0, :EMB_NF].set(fb)

    g32 = _gather_call(nf16, row3, col3, eaT)
    x4 = g32.reshape(E_PAD // 4, H_NF)
    w1big_bf = w1big.astype(_BF16)
    w2_bf = eW2.astype(_BF16)
    ef0 = _edge_mlp_call(x4, w1big_bf, e_b1, w2_bf, e_b2, 0, NB // 2)
    ef1 = _edge_mlp_call(x4, w1big_bf, e_b1, w2_bf, e_b2, NB // 2, NB // 2)
    zeros_big = jnp.zeros((N_PAD, H_NF), _F32)
    p00, p01 = _scatter_call(ef0, row4p[0], zeros_big, EPW // 2, NCH // 2)
    p10, p11 = _scatter_call(ef1, row4p[1], zeros_big, EPW // 2, NCH // 2)
    out8 = _node_mlp_call(nf16, p00, p01, p10, p11,
                          w1n, w1a, n_b1, nW2, n_b2, fw8, fb8)
    return out8[:, :EMB_NF]


# speedup vs baseline: 2.7647x; 1.0173x over previous
import functools

import jax
import jax.numpy as jnp
from jax import lax
from jax.experimental import pallas as pl
from jax.experimental.pallas import tpu as pltpu
from jax.experimental.pallas import tpu_sc as plsc

N_NODES = 10000
N_EDGES = 320000
NODE_NF = 11
EDGE_NF = 4
H_NF = 128
EMB_NF = 4

NC = 2
NS = 16
NW = NC * NS

CH = 128
EPW = 10240
NCH = EPW // CH
E_PAD = EPW * NW
N_PAD = 10112
RPT = N_PAD // NS

_F32 = jnp.float32


def _sc_mesh():
    return plsc.VectorSubcoreMesh(
        core_axis_name="c", subcore_axis_name="s", num_cores=NC, num_subcores=NS
    )


RG = 12
GLEAD = 10


def _gather_call(nf16, row3, col3, eaT):
    @functools.partial(
        pl.kernel,
        out_type=jax.ShapeDtypeStruct((E_PAD, 32), _F32),
        mesh=_sc_mesh(),
        scratch_types=[
            pltpu.VMEM((NCH, CH), jnp.int32),
            pltpu.VMEM((NCH, CH), jnp.int32),
            pltpu.VMEM((RG, CH, 16), _F32),
            pltpu.VMEM((RG, CH, 16), _F32),
            pltpu.VMEM((RG, EDGE_NF, CH), _F32),
            pltpu.VMEM_SHARED((N_PAD, 16), _F32),
            pltpu.SemaphoreType.DMA,
            pltpu.SemaphoreType.DMA,
            pltpu.SemaphoreType.DMA,
            pltpu.SemaphoreType.DMA,
        ],
        compiler_params=pltpu.CompilerParams(use_tc_tiling_on_sc=False,
                                             needs_layout_passes=False),
    )
    def k(nf_hbm, row_hbm, col_hbm, ea_hbm, g_hbm,
          ridx2, cidx2, sbuf, dbuf, abuf, nf_sp, gsem_r, gsem_c, gsem_a, wsem):
        s = lax.axis_index("s")
        wid = lax.axis_index("c") * NS + s

        tslice = pl.ds(s * RPT, RPT)
        pltpu.sync_copy(nf_hbm.at[tslice], nf_sp.at[tslice])

        pltpu.sync_copy(row_hbm.at[wid], ridx2)
        pltpu.sync_copy(col_hbm.at[wid], cidx2)
        plsc.subcore_barrier()

        lane = lax.iota(jnp.int32, 16)

        def ebase(t):
            return pl.multiple_of(wid * EPW + t * CH, CH)

        def start_gather(t, b):
            pltpu.async_copy(nf_sp.at[ridx2.at[t]], sbuf.at[b], gsem_r)
            pltpu.async_copy(nf_sp.at[cidx2.at[t]], dbuf.at[b], gsem_c)
            pltpu.async_copy(ea_hbm.at[:, pl.ds(ebase(t), CH)], abuf.at[b],
                             gsem_a)

        def wait_gather(t, b):
            pltpu.make_async_copy(nf_sp.at[ridx2.at[t]], sbuf.at[b], gsem_r).wait()
            pltpu.make_async_copy(nf_sp.at[cidx2.at[t]], dbuf.at[b], gsem_c).wait()
            pltpu.make_async_copy(ea_hbm.at[:, pl.ds(ebase(t), CH)], abuf.at[b],
                                  gsem_a).wait()

        def merge_ea(b):
            for c in range(EDGE_NF):
                cols = jnp.full((16,), NODE_NF + c, jnp.int32)
                for v in range(CH // 16):
                    vec = abuf[b, c, pl.ds(16 * v, 16)]
                    plsc.store_scatter(sbuf.at[b], [lane + 16 * v, cols], vec)

        def out_slices(t):
            base = ebase(t)
            return (g_hbm.at[pl.ds(base, CH), pl.ds(0, 16)],
                    g_hbm.at[pl.ds(base, CH), pl.ds(16, 16)])

        def start_write(t, b):
            o_s, o_d = out_slices(t)
            pltpu.async_copy(sbuf.at[b], o_s, wsem)
            pltpu.async_copy(dbuf.at[b], o_d, wsem)

        def wait_write(t, b):
            o_s, o_d = out_slices(t)
            pltpu.make_async_copy(sbuf.at[b], o_s, wsem).wait()
            pltpu.make_async_copy(dbuf.at[b], o_d, wsem).wait()

        for t in range(GLEAD):
            start_gather(t, t % RG)

        def body(g, carry):
            for b_off in range(RG):
                t = g * RG + b_off
                b = b_off
                bw = (b_off - 2) % RG

                @pl.when(t >= 2)
                def _():
                    wait_write(t - 2, bw)

                @pl.when(t + GLEAD < NCH)
                def _():
                    start_gather(t + GLEAD, bw)

                wait_gather(t, b)
                merge_ea(b)
                start_write(t, b)
            return carry

        lax.fori_loop(0, NCH // RG, body, 0, unroll=False)
        for t in range(NCH - NCH % RG, NCH):
            b = t % RG
            bw = (b - 2) % RG
            wait_write(t - 2, bw)

            @pl.when(t + GLEAD < NCH)
            def _():
                start_gather(t + GLEAD, bw)

            wait_gather(t, b)
            merge_ea(b)
            start_write(t, b)
        wait_write(NCH - 2, (NCH - 2) % RG)
        wait_write(NCH - 1, (NCH - 1) % RG)

    return k(nf16, row3, col3, eaT)


RS = 2
SLEAD = 1


def _scatter_call(ef, row3, zeros_big, epw, nch):
    @functools.partial(
        pl.kernel,
        out_type=(
            jax.ShapeDtypeStruct((N_PAD, H_NF), _F32),
            jax.ShapeDtypeStruct((N_PAD, H_NF), _F32),
        ),
        mesh=_sc_mesh(),
        scratch_types=[
            pltpu.VMEM((nch, CH), jnp.int32),
            pltpu.VMEM((RS, CH, H_NF), _F32),
            pltpu.VMEM_SHARED((N_PAD, H_NF), _F32),
            pltpu.SemaphoreType.DMA,
            pltpu.SemaphoreType.DMA,
        ],
        compiler_params=pltpu.CompilerParams(use_tc_tiling_on_sc=False),
    )
    def k(ef_hbm, row_hbm, z_hbm, p0_hbm, p1_hbm, idx2, ebuf, agg_sh,
          lsem, asem):
        c = lax.axis_index("c")
        s = lax.axis_index("s")
        wid = c * NS + s
        rslice = pl.ds(s * RPT, RPT)
        pltpu.sync_copy(z_hbm.at[rslice], agg_sh.at[rslice])
        pltpu.sync_copy(row_hbm.at[wid], idx2)
        plsc.subcore_barrier()

        def _ef_pairs(t, b):
            base = wid * epw + t * CH
            blk = base // BE
            i0 = pl.multiple_of((base % BE) // 4, CH // 4)
            return [(ef_hbm.at[blk, j, pl.ds(i0, CH // 4)],
                     ebuf.at[b, pl.ds(j * (CH // 4), CH // 4)])
                    for j in range(4)]

        def start_load(t, b):
            for src, dst in _ef_pairs(t, b):
                pltpu.async_copy(src, dst, lsem)

        def wait_load(t, b):
            for src, dst in _ef_pairs(t, b):
                pltpu.make_async_copy(src, dst, lsem).wait()

        def start_add(t, b):
            pltpu.async_copy(ebuf.at[b], agg_sh.at[idx2.at[t]], asem, add=True)

        def wait_add(t, b):
            pltpu.make_async_copy(ebuf.at[b], agg_sh.at[idx2.at[t]], asem).wait()

        start_load(0, 0)

        def body(g, carry):
            for b in range(RS):
                t = g * RS + b
                bo = 1 - b

                @pl.when(t >= 1)
                def _():
                    wait_add(t - 1, bo)

                @pl.when(t + 1 < nch)
                def _():
                    start_load(t + 1, bo)

                wait_load(t, b)
                start_add(t, b)
            return carry

        lax.fori_loop(0, nch // RS, body, 0, unroll=False)
        wait_add(nch - 1, (nch - 1) % RS)
        plsc.subcore_barrier()

        @pl.when(c == 0)
        def _():
            pltpu.sync_copy(agg_sh.at[rslice], p0_hbm.at[rslice])

        @pl.when(c == 1)
        def _():
            pltpu.sync_copy(agg_sh.at[rslice], p1_hbm.at[rslice])

    return k(ef, row3, zeros_big)


BE = 2048


_BF16 = jnp.bfloat16


BE4 = BE // 4
NB = E_PAD // BE


def _edge_mlp_kernel(x4, w1big, b1, w2, b2, out):
    xb = x4[...].astype(_BF16)
    for j in range(4):
        h = jnp.dot(xb, w1big[j], preferred_element_type=_F32)
        h = jnp.maximum(h + b1[...], 0.0)
        h = jnp.dot(h.astype(_BF16), w2[...],
                    preferred_element_type=_F32) + b2[...]
        out[0, j] = jnp.maximum(h, 0.0)


def _edge_mlp_call(x4, w1big, b1, w2, b2, blk0, nb):
    bcast = lambda shape: pl.BlockSpec(shape, lambda i: tuple(0 for _ in shape))
    return pl.pallas_call(
        _edge_mlp_kernel,
        grid=(nb,),
        in_specs=[
            pl.BlockSpec((BE4, H_NF), lambda i: (i + blk0, 0)),
            bcast((4, H_NF, H_NF)),
            bcast((1, H_NF)),
            bcast((H_NF, H_NF)),
            bcast((1, H_NF)),
        ],
        out_specs=pl.BlockSpec((1, 4, BE4, H_NF), lambda i: (i, 0, 0, 0)),
        out_shape=jax.ShapeDtypeStruct((nb, 4, BE4, H_NF), _F32),
    )(x4, w1big, b1, w2, b2)


def _permute_idx_kernel(idx2d, perm, out):
    xf = idx2d[...].astype(_F32)
    pf = perm[...]
    out[...] = jnp.dot(xf, pf, preferred_element_type=_F32,
                       precision=lax.Precision.HIGHEST).astype(jnp.int32)


def _permute_idx_call(idx2d, perm):
    nrows = E_PAD // CH
    return pl.pallas_call(
        _permute_idx_kernel,
        grid=(2,),
        in_specs=[
            pl.BlockSpec((nrows // 2, CH), lambda i: (i, 0)),
            pl.BlockSpec((CH, CH), lambda i: (0, 0)),
        ],
        out_specs=pl.BlockSpec((nrows // 2, CH), lambda i: (i, 0)),
        out_shape=jax.ShapeDtypeStruct((nrows, CH), jnp.int32),
    )(idx2d, perm)


BN = 1024


NPARTS = 8


def _node_mlp_kernel(nf, *args):
    parts = args[:NPARTS]
    w1n, w1a, b1, w2, b2, fw, fb, out = args[NPARTS:]
    agg = parts[0][...]
    for p in parts[1:]:
        agg = agg + p[...]
    h = jnp.dot(nf[...], w1n[...], preferred_element_type=_F32)
    h = h + jnp.dot(agg, w1a[...], preferred_element_type=_F32)
    h = jnp.maximum(h + b1[...], 0.0)
    h = jnp.dot(h, w2[...], preferred_element_type=_F32) + b2[...]
    out[...] = jnp.dot(h, fw[...], preferred_element_type=_F32) + fb[...]


def _node_mlp_call(nf16, parts, w1n, w1a, b1, w2, b2, fw8, fb8):
    grid = (pl.cdiv(N_NODES, BN),)
    bcast = lambda shape: pl.BlockSpec(shape, lambda i: (0, 0))
    return pl.pallas_call(
        _node_mlp_kernel,
        grid=grid,
        in_specs=[
            pl.BlockSpec((BN, 16), lambda i: (i, 0)),
            *[pl.BlockSpec((BN, H_NF), lambda i: (i, 0))
              for _ in range(NPARTS)],
            bcast((16, H_NF)),
            bcast((H_NF, H_NF)),
            bcast((1, H_NF)),
            bcast((H_NF, H_NF)),
            bcast((1, H_NF)),
            bcast((H_NF, 8)),
            bcast((1, 8)),
        ],
        out_specs=pl.BlockSpec((BN, 8), lambda i: (i, 0)),
        out_shape=jax.ShapeDtypeStruct((N_NODES, 8), _F32),
    )(nf16, *parts, w1n, w1a, b1, w2, b2, fw8, fb8)


def kernel(node_feats, edge_index, edge_attr,
           eW1, eb1, eW2, eb2, nW1, nb1, nW2, nb2, fW, fb):
    row = edge_index[0]
    col = edge_index[1]
    pad_idx = jnp.full((E_PAD - N_EDGES,), N_NODES, jnp.int32)
    row_pad = jnp.concatenate([row, pad_idx])
    row3 = row_pad.reshape(NW, NCH, CH)
    col3 = jnp.concatenate([col, pad_idx]).reshape(NW, NCH, CH)
    src_of = 4 * (jnp.arange(CH) % (CH // 4)) + jnp.arange(CH) // (CH // 4)
    perm = jnp.zeros((CH, CH), _F32).at[src_of, jnp.arange(CH)].set(1.0)
    row4p = _permute_idx_call(row_pad.reshape(E_PAD // CH, CH),
                              perm).reshape(4, NW, NCH // 4, CH)

    nf16 = jnp.zeros((N_PAD, 16), _F32).at[:N_NODES, :NODE_NF].set(node_feats)
    eaT = jnp.pad(edge_attr.T, ((0, 0), (0, E_PAD - N_EDGES)))

    w1sd = jnp.zeros((32, H_NF), _F32)
    w1sd = w1sd.at[:NODE_NF].set(eW1[:NODE_NF])
    w1sd = w1sd.at[NODE_NF:NODE_NF + EDGE_NF].set(eW1[2 * NODE_NF:])
    w1sd = w1sd.at[16:16 + NODE_NF].set(eW1[NODE_NF:2 * NODE_NF])
    w1big = jnp.zeros((4, H_NF, H_NF), _F32)
    for j in range(4):
        w1big = w1big.at[j, 32 * j:32 * j + 32].set(w1sd)
    e_b1 = eb1.reshape(1, H_NF)
    e_b2 = eb2.reshape(1, H_NF)

    w1n = jnp.zeros((16, H_NF), _F32).at[:NODE_NF].set(nW1[:NODE_NF])
    w1a = nW1[NODE_NF:]
    n_b1 = nb1.reshape(1, H_NF)
    n_b2 = nb2.reshape(1, H_NF)
    fw8 = jnp.zeros((H_NF, 8), _F32).at[:, :EMB_NF].set(fW)
    fb8 = jnp.zeros((1, 8), _F32).at[0, :EMB_NF].set(fb)

    g32 = _gather_call(nf16, row3, col3, eaT)
    x4 = g32.reshape(E_PAD // 4, H_NF)
    w1big_bf = w1big.astype(_BF16)
    w2_bf = eW2.astype(_BF16)
    zeros_big = jnp.zeros((N_PAD, H_NF), _F32)
    parts = []
    for h in range(4):
        ef_h = _edge_mlp_call(x4, w1big_bf, e_b1, w2_bf, e_b2,
                              h * (NB // 4), NB // 4)
        parts.extend(_scatter_call(ef_h, row4p[h], zeros_big,
                                   EPW // 4, NCH // 4))
    out8 = _node_mlp_call(nf16, parts,
                          w1n, w1a, n_b1, nW2, n_b2, fw8, fb8)
    return out8[:, :EMB_NF]
